# Initial kernel scaffold; baseline (speedup 1.0000x reference)
import jax
import jax.numpy as jnp
from jax.experimental import pallas as pl


def kernel(x, edge_index, Wq, Wk, Wv, Wo, bo, W1, b1, W2, b2, g1, be1, g2, be2):
    raise NotImplementedError("write your pallas kernel here")


# speedup vs baseline: 17.1275x; 17.1275x over previous
import functools

import jax
import jax.numpy as jnp
import numpy as np
from jax import lax
from jax.experimental import pallas as pl
from jax.experimental.pallas import tpu as pltpu
from jax.experimental.pallas import tpu_sc as plsc

_N, _E, _D, _H = 10000, 320000, 128, 8
_DH = _D // _H
_NC, _NS = 2, 16
_NW = _NC * _NS
_EPW = _E // _NW
_C = 80
_NCHUNK = _EPW // _C
_NPAD = 10240
_XROWS = _NPAD // 8
_AROWS = _NPAD + _XROWS
_RPT = _AROWS // _NS
_PREC = lax.Precision.HIGHEST


def _qkv_body(x_ref, wq_ref, wk_ref, wv_ref, q_ref, k_ref, v_ref):
    x = x_ref[...]
    q_ref[...] = jnp.dot(x, wq_ref[...], preferred_element_type=jnp.float32,
                         precision=_PREC)
    k_ref[...] = jnp.dot(x, wk_ref[...], preferred_element_type=jnp.float32,
                         precision=_PREC)
    v_ref[...] = jnp.dot(x, wv_ref[...], preferred_element_type=jnp.float32,
                         precision=_PREC)


_qkv_call = pl.pallas_call(
    _qkv_body,
    out_shape=[jax.ShapeDtypeStruct((_N, _D), jnp.float32)] * 3,
)


_mesh = plsc.VectorSubcoreMesh(core_axis_name="c", subcore_axis_name="s")


@functools.partial(
    pl.kernel,
    out_type=jax.ShapeDtypeStruct((_NC, _AROWS, _D), jnp.float32),
    mesh=_mesh,
    scratch_types=[
        pltpu.VMEM((_C,), jnp.int32),
        pltpu.VMEM((_C,), jnp.int32),
        pltpu.VMEM((_C,), jnp.int32),
        pltpu.VMEM((_C, _D), jnp.float32),
        pltpu.VMEM((_C, _D), jnp.float32),
        pltpu.VMEM((_C, _D), jnp.float32),
        pltpu.VMEM_SHARED((_AROWS, _D), jnp.float32),
        pltpu.SemaphoreType.DMA,
        pltpu.SemaphoreType.DMA,
        pltpu.SemaphoreType.DMA,
    ],
)
def _edge_kernel(q_hbm, k_hbm, v_hbm, src_hbm, dst_hbm, out_hbm,
                 dst_v, src_v, xrow_v, q_v, k_v, v_v,
                 accum, sem_q, sem_k, sem_v):
    cid = lax.axis_index("c")
    sid = lax.axis_index("s")
    wid = sid * _NC + cid
    ebase = wid * _EPW

    zeros16 = jnp.zeros((16,), jnp.float32)

    def zrow(i, _):
        for j in range(_D // 16):
            q_v[i, pl.ds(j * 16, 16)] = zeros16
        return 0

    lax.fori_loop(0, _C, zrow, 0)
    for r in range(_RPT // _C):
        pltpu.sync_copy(q_v, accum.at[pl.ds(sid * _RPT + r * _C, _C)])
    plsc.subcore_barrier()

    lanes = lax.iota(jnp.int32, 16)
    scale = np.float32(1.0 / np.sqrt(_DH))

    def chunk_body(i, _):
        pltpu.sync_copy(dst_hbm.at[pl.ds(ebase + i * _C, _C)], dst_v)
        pltpu.sync_copy(src_hbm.at[pl.ds(ebase + i * _C, _C)], src_v)

        def cp(j, _):
            dv = dst_v[pl.ds(j * 16, 16)]
            xrow_v[pl.ds(j * 16, 16)] = _NPAD + lax.shift_right_logical(dv, 3)
            return 0

        lax.fori_loop(0, _C // 16, cp, 0, unroll=True)

        cq = pltpu.async_copy(q_hbm.at[dst_v], q_v, sem_q)
        ck = pltpu.async_copy(k_hbm.at[src_v], k_v, sem_k)
        cv = pltpu.async_copy(v_hbm.at[src_v], v_v, sem_v)
        cq.wait()
        ck.wait()
        cv.wait()

        def group_body(g, _):
            b16 = jnp.bitwise_and(dst_v[pl.ds(g * 16, 16)], 7)

            def edge_body(j, _):
                e = g * 16 + j
                ex16 = jnp.zeros((16,), jnp.float32)
                for h in range(_H):
                    q = q_v[e, pl.ds(h * _DH, _DH)]
                    k = k_v[e, pl.ds(h * _DH, _DH)]
                    p = q * k
                    for sh in (8, 4, 2, 1):
                        p = p + p[lanes ^ sh]
                    s = p * scale
                    s = jnp.minimum(jnp.maximum(s, np.float32(-5.0)),
                                    np.float32(5.0))
                    ev = jnp.exp(s)
                    v = v_v[e, pl.ds(h * _DH, _DH)]
                    q_v[e, pl.ds(h * _DH, _DH)] = v * ev
                    ex16 = jnp.where(lanes == h, ev, ex16)
                m = jnp.where(lanes == j, b16, 0)
                for sh in (8, 4, 2, 1):
                    m = m + m[lanes ^ sh]
                for blk in range(8):
                    k_v[e, pl.ds(blk * _DH, 16)] = jnp.where(
                        m == blk, ex16, zeros16)
                return 0

            lax.fori_loop(0, 16, edge_body, 0)
            return 0

        lax.fori_loop(0, _C // 16, group_body, 0)

        pltpu.sync_copy(q_v, accum.at[dst_v], add=True)
        pltpu.sync_copy(k_v, accum.at[xrow_v], add=True)
        return 0

    lax.fori_loop(0, _NCHUNK, chunk_body, 0)

    plsc.subcore_barrier()
    pltpu.sync_copy(accum.at[pl.ds(sid * _RPT, _RPT)],
                    out_hbm.at[cid, pl.ds(sid * _RPT, _RPT)])


def _post_body(x_ref, p0_ref, p1_ref, e0_ref, e1_ref, wo_ref, bo_ref,
               w1_ref, b1_ref, w2_ref, b2_ref, g1_ref, be1_ref, g2_ref,
               be2_ref, o_ref):
    msg = p0_ref[...] + p1_ref[...]
    ssum = e0_ref[:, :_H] + e1_ref[:, :_H]
    recip = 1.0 / (ssum + 1e-16)
    sel = (lax.broadcasted_iota(jnp.int32, (_H, _D), 0)
           == lax.broadcasted_iota(jnp.int32, (_H, _D), 1) // _DH
           ).astype(jnp.float32)
    expand = jnp.dot(recip, sel, preferred_element_type=jnp.float32,
                     precision=_PREC)
    aggr = msg * expand

    x = x_ref[...]
    h = jnp.dot(aggr, wo_ref[...], preferred_element_type=jnp.float32,
                precision=_PREC) + bo_ref[...] + x
    mu = jnp.mean(h, axis=0)
    var = jnp.mean((h - mu) ** 2, axis=0)
    h = (h - mu) * lax.rsqrt(var + 1e-5) * g1_ref[...] + be1_ref[...]
    h_in2 = h
    h = jnp.dot(h, w1_ref[...], preferred_element_type=jnp.float32,
                precision=_PREC) + b1_ref[...]
    h = jnp.maximum(h, 0.0)
    h = jnp.dot(h, w2_ref[...], preferred_element_type=jnp.float32,
                precision=_PREC) + b2_ref[...]
    h = h_in2 + h
    mu = jnp.mean(h, axis=0)
    var = jnp.mean((h - mu) ** 2, axis=0)
    o_ref[...] = (h - mu) * lax.rsqrt(var + 1e-5) * g2_ref[...] + be2_ref[...]


_post_call = pl.pallas_call(
    _post_body,
    out_shape=jax.ShapeDtypeStruct((_N, _D), jnp.float32),
    compiler_params=pltpu.CompilerParams(vmem_limit_bytes=100 * 1024 * 1024),
)


def kernel(x, edge_index, Wq, Wk, Wv, Wo, bo, W1, b1, W2, b2, g1, be1, g2,
           be2):
    q, k, v = _qkv_call(x, Wq, Wk, Wv)
    src = edge_index[0]
    dst = edge_index[1]
    partials = _edge_kernel(q, k, v, src, dst)
    ex0 = partials[0, _NPAD:].reshape(_NPAD, 16)[:_N]
    ex1 = partials[1, _NPAD:].reshape(_NPAD, 16)[:_N]
    return _post_call(x, partials[0, :_N], partials[1, :_N], ex0, ex1, Wo,
                      bo, W1, b1, W2, b2, g1, be1, g2, be2)
